# Initial kernel scaffold; baseline (speedup 1.0000x reference)
import jax
import jax.numpy as jnp
from jax.experimental import pallas as pl


def kernel(x, table):
    raise NotImplementedError("write your pallas kernel here")


# speedup vs baseline: 83.9837x; 83.9837x over previous
import functools

import jax
import jax.numpy as jnp
import numpy as np
from jax import lax
from jax.experimental import pallas as pl
from jax.experimental.pallas import tpu as pltpu
from jax.experimental.pallas import tpu_sc as plsc

DIM = 16
NUM_CLASSES = 100000
DELTA = np.float32(1.0) / np.float32(100000.0)
SCALE = np.float32(100000.0)

CHUNK = 1024
GFAN = CHUNK // 128
LANES = 16


def _bin_index(xx):
    k0 = (xx * SCALE).astype(jnp.int32)
    kk = k0 + 3
    for off in (2, 1, 0, -1, -2):
        kc = k0 + off
        gv = kc.astype(jnp.float32) * DELTA
        kk = jnp.where(gv >= xx, kc, kk)
    kk = jnp.minimum(jnp.maximum(kk, 0), NUM_CLASSES)
    return kk + 2


def _make_sc_kernel(B, n_rows):
    info = plsc.get_sparse_core_info()
    NC, NS = info.num_cores, info.num_subcores
    NW = NC * NS
    assert B % (NW * CHUNK) == 0
    b_per_w = B // NW
    n_chunks = b_per_w // CHUNK
    mesh = plsc.VectorSubcoreMesh(core_axis_name="c", subcore_axis_name="s")

    @functools.partial(
        pl.kernel,
        mesh=mesh,
        out_type=jax.ShapeDtypeStruct((B, DIM), jnp.float32),
        scratch_types=[
            pltpu.VMEM((CHUNK,), jnp.float32),
            pltpu.VMEM((GFAN, 128), jnp.int32),
            pltpu.VMEM((CHUNK, DIM), jnp.float32),
            pltpu.SemaphoreType.DMA,
        ],
        compiler_params=pltpu.CompilerParams(use_tc_tiling_on_sc=False),
    )
    def sc_embed(x_hbm, table_hbm, out_hbm, xv, idxv, rowsv, sem):
        wid = lax.axis_index("s") * NC + lax.axis_index("c")
        base = wid * b_per_w

        def chunk_body(g, carry):
            start = base + g * CHUNK
            pltpu.sync_copy(x_hbm.at[pl.ds(start, CHUNK)], xv)

            def vec_body(i, carry2):
                j = i // (128 // LANES)
                l = i - j * (128 // LANES)
                xx = xv[pl.ds(i * LANES, LANES)]
                idxv[j, pl.ds(l * LANES, LANES)] = _bin_index(xx)
                return carry2

            lax.fori_loop(0, CHUNK // LANES, vec_body, 0)

            copies = [
                pltpu.async_copy(
                    table_hbm.at[idxv.at[j]],
                    rowsv.at[pl.ds(j * 128, 128)],
                    sem,
                )
                for j in range(GFAN)
            ]
            for c in copies:
                c.wait()
            pltpu.sync_copy(rowsv, out_hbm.at[pl.ds(start, CHUNK)])
            return carry

        lax.fori_loop(0, n_chunks, chunk_body, 0)

    return sc_embed


def kernel(x, table):
    R, C = x.shape
    B = R * C
    out = _make_sc_kernel(B, table.shape[0])(x.reshape(B), table)
    return out.reshape(R, C, DIM)


# speedup vs baseline: 284.6829x; 3.3897x over previous
import functools

import jax
import jax.numpy as jnp
import numpy as np
from jax import lax
from jax.experimental import pallas as pl
from jax.experimental.pallas import tpu as pltpu
from jax.experimental.pallas import tpu_sc as plsc

DIM = 16
NUM_CLASSES = 100000
DELTA = np.float32(1.0) / np.float32(100000.0)
SCALE = np.float32(100000.0)

LANES = 16
TR_MODE = False
RPC = 16
XCOLS = 100
CHUNK = RPC * XCOLS


def _bin_index(xx):
    k0 = (xx * SCALE).astype(jnp.int32)
    kk = k0 + 3
    for off in (2, 1, 0, -1, -2):
        kc = k0 + off
        gv = kc.astype(jnp.float32) * DELTA
        kk = jnp.where(gv >= xx, kc, kk)
    kk = jnp.minimum(jnp.maximum(kk, 0), NUM_CLASSES)
    return kk + 2


def _make_sc_kernel(R, n_rows):
    info = plsc.get_sparse_core_info()
    NC, NS = info.num_cores, info.num_subcores
    NW = NC * NS
    assert R % (NW * RPC) == 0
    rows_per_w = R // NW
    n_chunks = rows_per_w // RPC
    segs = []
    off = 0
    while off < CHUNK:
        seg = min(128, CHUNK - off)
        segs.append((off, seg))
        off += seg
    mesh = plsc.VectorSubcoreMesh(core_axis_name="c", subcore_axis_name="s")

    @functools.partial(
        pl.kernel,
        mesh=mesh,
        out_type=jax.ShapeDtypeStruct((DIM, R, XCOLS), jnp.float32),
        scratch_types=[
            pltpu.VMEM((CHUNK,), jnp.float32),
            pltpu.VMEM((CHUNK,), jnp.int32),
            pltpu.VMEM((CHUNK, DIM), jnp.float32),
            pltpu.VMEM((DIM, RPC, XCOLS), jnp.float32),
            pltpu.SemaphoreType.DMA,
        ],
        compiler_params=pltpu.CompilerParams(use_tc_tiling_on_sc=False, needs_layout_passes=False),
    )
    def sc_embed(x_hbm, table_hbm, out_hbm, xv, idxv, rowsv, outtv, sem):
        wid = lax.axis_index("s") * NC + lax.axis_index("c")
        row_base = wid * rows_per_w
        lane = lax.iota(jnp.int32, LANES)
        fvecs = [jnp.full((LANES,), f, jnp.int32) for f in range(DIM)]

        def chunk_body(g, carry):
            row0 = row_base + g * RPC
            s0 = row0 * XCOLS
            pltpu.sync_copy(x_hbm.at[pl.ds(s0, CHUNK)], xv)

            def bin_body(b, c2):
                xx = xv[pl.ds(b * LANES, LANES)]
                idxv[pl.ds(b * LANES, LANES)] = _bin_index(xx)
                return c2

            lax.fori_loop(0, CHUNK // LANES, bin_body, 0)

            copies = [
                pltpu.async_copy(
                    table_hbm.at[idxv.at[pl.ds(o, n)]],
                    rowsv.at[pl.ds(o, n)],
                    sem,
                )
                for o, n in segs
            ]
            for c in copies:
                c.wait()

            def tr_body(b, c2):
                p = b * LANES + lane
                r = p // XCOLS
                col = p - r * XCOLS
                for f in range(DIM):
                    vals = plsc.load_gather(rowsv, [p, fvecs[f]])
                    plsc.store_scatter(outtv, [fvecs[f], r, col], vals)
                return c2

            lax.fori_loop(0, CHUNK // LANES, tr_body, 0)

            for f in range(DIM):
                pltpu.sync_copy(
                    outtv.at[f], out_hbm.at[f, pl.ds(row0, RPC)]
                )
            return carry

        lax.fori_loop(0, n_chunks, chunk_body, 0)

    return sc_embed


def kernel(x, table):
    R, C = x.shape
    out = _make_sc_kernel(R, table.shape[0])(x.reshape(R * C), table)
    return out.transpose(1, 2, 0)


# speedup vs baseline: 291.7963x; 1.0250x over previous
import functools

import jax
import jax.numpy as jnp
import numpy as np
from jax import lax
from jax.experimental import pallas as pl
from jax.experimental.pallas import tpu as pltpu
from jax.experimental.pallas import tpu_sc as plsc

DIM = 16
NUM_CLASSES = 100000
DELTA = np.float32(1.0) / np.float32(100000.0)
SCALE = np.float32(100000.0)

LANES = 16
TR_MODE = False
RPC = 16
XCOLS = 100
CHUNK = RPC * XCOLS


def _bin_index(xx):
    k0 = (xx * SCALE).astype(jnp.int32)
    kk = k0 + 3
    for off in (2, 1, 0, -1, -2):
        kc = k0 + off
        gv = kc.astype(jnp.float32) * DELTA
        kk = jnp.where(gv >= xx, kc, kk)
    kk = jnp.minimum(jnp.maximum(kk, 0), NUM_CLASSES)
    return kk + 2


def _make_sc_kernel(R, n_rows):
    info = plsc.get_sparse_core_info()
    NC, NS = info.num_cores, info.num_subcores
    NW = NC * NS
    assert R % (NW * RPC) == 0
    rows_per_w = R // NW
    n_chunks = rows_per_w // RPC
    segs = []
    off = 0
    while off < CHUNK:
        seg = min(128, CHUNK - off)
        segs.append((off, seg))
        off += seg
    mesh = plsc.VectorSubcoreMesh(core_axis_name="c", subcore_axis_name="s")

    @functools.partial(
        pl.kernel,
        mesh=mesh,
        out_type=jax.ShapeDtypeStruct((DIM, R, XCOLS), jnp.float32),
        scratch_types=[
            pltpu.VMEM((CHUNK,), jnp.float32),
            pltpu.VMEM((CHUNK,), jnp.int32),
            pltpu.VMEM((CHUNK, DIM), jnp.float32),
            pltpu.VMEM((DIM, RPC, XCOLS), jnp.float32),
            pltpu.SemaphoreType.DMA,
        ],
        compiler_params=pltpu.CompilerParams(use_tc_tiling_on_sc=False, needs_layout_passes=False),
    )
    def sc_embed(x_hbm, table_hbm, out_hbm, xv, idxv, rowsv, outtv, sem):
        wid = lax.axis_index("s") * NC + lax.axis_index("c")
        row_base = wid * rows_per_w
        lane = lax.iota(jnp.int32, LANES)
        fvecs = [jnp.full((LANES,), f, jnp.int32) for f in range(DIM)]

        def chunk_body(g, carry):
            row0 = row_base + g * RPC
            s0 = row0 * XCOLS
            pltpu.sync_copy(x_hbm.at[pl.ds(s0, CHUNK)], xv)

            def bin_body(b, c2):
                xx = xv[pl.ds(b * LANES, LANES)]
                idxv[pl.ds(b * LANES, LANES)] = _bin_index(xx)
                return c2

            lax.fori_loop(0, CHUNK // LANES, bin_body, 0)

            copies = [
                pltpu.async_copy(
                    table_hbm.at[idxv.at[pl.ds(o, n)]],
                    rowsv.at[pl.ds(o, n)],
                    sem,
                )
                for o, n in segs
            ]
            for c in copies:
                c.wait()

            def tr_body(b, c2):
                p = b * LANES + lane
                r = p // XCOLS
                col = p - r * XCOLS
                for f in range(DIM):
                    vals = plsc.load_gather(rowsv, [p, fvecs[f]])
                    plsc.store_scatter(outtv, [fvecs[f], r, col], vals)
                return c2

            lax.fori_loop(0, CHUNK // LANES, tr_body, 0)

            pltpu.sync_copy(outtv, out_hbm.at[:, pl.ds(row0, RPC)])
            return carry

        lax.fori_loop(0, n_chunks, chunk_body, 0)

    return sc_embed


def kernel(x, table):
    R, C = x.shape
    out = _make_sc_kernel(R, table.shape[0])(x.reshape(R * C), table)
    return out.transpose(1, 2, 0)


# speedup vs baseline: 308.1068x; 1.0559x over previous
import functools

import jax
import jax.numpy as jnp
import numpy as np
from jax import lax
from jax.experimental import pallas as pl
from jax.experimental.pallas import tpu as pltpu
from jax.experimental.pallas import tpu_sc as plsc

DIM = 16
NUM_CLASSES = 100000
DELTA = np.float32(1.0) / np.float32(100000.0)
SCALE = np.float32(100000.0)

LANES = 16
TR_MODE = False
RPC = 16
XCOLS = 100
CHUNK = RPC * XCOLS


def _bin_index(xx):
    k0 = (xx * SCALE).astype(jnp.int32)
    kk = k0 + 3
    for off in (2, 1, 0, -1, -2):
        kc = k0 + off
        gv = kc.astype(jnp.float32) * DELTA
        kk = jnp.where(gv >= xx, kc, kk)
    kk = jnp.minimum(jnp.maximum(kk, 0), NUM_CLASSES)
    return kk + 2


def _make_sc_kernel(R, n_rows):
    info = plsc.get_sparse_core_info()
    NC, NS = info.num_cores, info.num_subcores
    NW = NC * NS
    assert R % (NW * RPC) == 0
    rows_per_w = R // NW
    n_chunks = rows_per_w // RPC
    segs = []
    off = 0
    while off < CHUNK:
        seg = min(128, CHUNK - off)
        segs.append((off, seg))
        off += seg
    mesh = plsc.VectorSubcoreMesh(core_axis_name="c", subcore_axis_name="s")

    @functools.partial(
        pl.kernel,
        mesh=mesh,
        out_type=jax.ShapeDtypeStruct((DIM, R, XCOLS), jnp.float32),
        scratch_types=[
            pltpu.VMEM((CHUNK,), jnp.float32),
            pltpu.VMEM((CHUNK,), jnp.int32),
            pltpu.VMEM((CHUNK, DIM), jnp.float32),
            pltpu.VMEM((DIM, RPC, XCOLS), jnp.float32),
            pltpu.SemaphoreType.DMA,
            pltpu.SemaphoreType.DMA,
        ],
        compiler_params=pltpu.CompilerParams(use_tc_tiling_on_sc=False, needs_layout_passes=False),
    )
    def sc_embed(x_hbm, table_hbm, out_hbm, xv, idxv, rowsv, outtv, sem, osem):
        wid = lax.axis_index("s") * NC + lax.axis_index("c")
        row_base = wid * rows_per_w
        lane = lax.iota(jnp.int32, LANES)
        fvecs = [jnp.full((LANES,), f, jnp.int32) for f in range(DIM)]

        def chunk_body(g, carry):
            row0 = row_base + g * RPC
            s0 = row0 * XCOLS
            pltpu.sync_copy(x_hbm.at[pl.ds(s0, CHUNK)], xv)

            def bin_body(b, c2):
                xx = xv[pl.ds(b * LANES, LANES)]
                idxv[pl.ds(b * LANES, LANES)] = _bin_index(xx)
                return c2

            lax.fori_loop(0, CHUNK // LANES, bin_body, 0)

            copies = [
                pltpu.async_copy(
                    table_hbm.at[idxv.at[pl.ds(o, n)]],
                    rowsv.at[pl.ds(o, n)],
                    sem,
                )
                for o, n in segs
            ]

            def tr_body(b, c2):
                p = b * LANES + lane
                mm = p * jnp.int32(5243)
                r = lax.shift_right_logical(mm, jnp.int32(19))
                col = p - r * XCOLS
                for f in range(DIM):
                    vals = plsc.load_gather(rowsv, [p, fvecs[f]])
                    plsc.store_scatter(outtv, [fvecs[f], r, col], vals)
                return c2

            @pl.when(g > 0)
            def _():
                pltpu.make_async_copy(
                    outtv, out_hbm.at[:, pl.ds(row0, RPC)], osem
                ).wait()

            for j, (o, n) in enumerate(segs):
                copies[j].wait()
                lax.fori_loop(o // LANES, (o + n) // LANES, tr_body, 0)

            pltpu.async_copy(outtv, out_hbm.at[:, pl.ds(row0, RPC)], osem)
            return carry

        lax.fori_loop(0, n_chunks, chunk_body, 0)
        last_row0 = row_base + (n_chunks - 1) * RPC
        pltpu.make_async_copy(
            outtv, out_hbm.at[:, pl.ds(last_row0, RPC)], osem
        ).wait()

    return sc_embed


def kernel(x, table):
    R, C = x.shape
    out = _make_sc_kernel(R, table.shape[0])(x.reshape(R * C), table)
    return out.transpose(1, 2, 0)
